# Initial kernel scaffold; baseline (speedup 1.0000x reference)
import jax
import jax.numpy as jnp
from jax.experimental import pallas as pl


def kernel(indices, special_weight, pretrained_weights):
    raise NotImplementedError("write your pallas kernel here")


# speedup vs baseline: 2.7350x; 2.7350x over previous
import functools

import jax
import jax.numpy as jnp
from jax import lax
from jax.experimental import pallas as pl
from jax.experimental.pallas import tpu as pltpu
from jax.experimental.pallas import tpu_sc as plsc

_NUM_SPECIAL = 4
_VOCAB = 1000000
_DIM = 64
_B_TOTAL = 4096 * 200
_NW = 32
_B_PER_W = _B_TOTAL // _NW
_CHUNK = 640
_K = _CHUNK // 128
_N_CHUNKS = _B_PER_W // _CHUNK
_L = 16


def _glove_body(idx_hbm, special_hbm, pret_hbm, out_hbm,
                idx_raw, pi_2d, rows, special_v, sem):
    wid = lax.axis_index("c") * 16 + lax.axis_index("s")
    base_w = wid * _B_PER_W

    pltpu.sync_copy(special_hbm, special_v)

    def chunk_body(ci, carry):
        base = base_w + ci * _CHUNK
        pltpu.sync_copy(idx_hbm.at[pl.ds(base, _CHUNK)], idx_raw)

        any_vec = jnp.zeros((_L,), jnp.int32)
        for i in range(_CHUNK // _L):
            v = idx_raw[pl.ds(i * _L, _L)]
            any_vec = any_vec | jnp.where(v < _NUM_SPECIAL, 1, 0).astype(jnp.int32)
            p = jnp.maximum(v - (_NUM_SPECIAL - 1), 0)
            p = jnp.minimum(p, _VOCAB + 1)
            pi_2d[i // 8, pl.ds((i % 8) * _L, _L)] = p
        n_special = any_vec[0]
        for t in range(1, _L):
            n_special = n_special | any_vec[t]

        copies = []
        for j in range(_K):
            copies.append(pltpu.async_copy(
                pret_hbm.at[pi_2d.at[j]],
                rows.at[pl.ds(j * 128, 128)],
                sem))
        for c in copies:
            c.wait()

        @pl.when(n_special > 0)
        def _fixup():
            def group_body(g, c2):
                v = idx_raw[pl.ds(g * _L, _L)]
                for j in range(_L):
                    ikj = v[j]

                    @pl.when(ikj < _NUM_SPECIAL)
                    def _fix(ikj=ikj, j=j):
                        row = g * _L + j
                        for c in range(_DIM // _L):
                            sl = pl.ds(c * _L, _L)
                            rows[row, sl] = rows[row, sl] + special_v[ikj, sl]
                return c2

            lax.fori_loop(0, _CHUNK // _L, group_body, 0)

        pltpu.sync_copy(rows, out_hbm.at[pl.ds(base, _CHUNK)])
        return carry

    lax.fori_loop(0, _N_CHUNKS, chunk_body, 0)


_mesh = plsc.VectorSubcoreMesh(core_axis_name="c", subcore_axis_name="s")

_glove_kernel = functools.partial(
    pl.kernel,
    mesh=_mesh,
    compiler_params=pltpu.CompilerParams(use_tc_tiling_on_sc=False),
    out_type=jax.ShapeDtypeStruct((_B_TOTAL, _DIM), jnp.float32),
    scratch_types=[
        pltpu.VMEM((_CHUNK,), jnp.int32),
        pltpu.VMEM((_K, 128), jnp.int32),
        pltpu.VMEM((_CHUNK, _DIM), jnp.float32),
        pltpu.VMEM((_NUM_SPECIAL + 1, _DIM), jnp.float32),
        pltpu.SemaphoreType.DMA,
    ],
)(_glove_body)


@jax.jit
def kernel(indices, special_weight, pretrained_weights):
    flat = indices.reshape(-1)
    out = _glove_kernel(flat, special_weight, pretrained_weights)
    return out.reshape(indices.shape + (_DIM,))
